# baseline (speedup 1.0000x reference)
import jax
import jax.numpy as jnp
from jax import lax
from jax.experimental import pallas as pl
from jax.experimental.pallas import tpu as pltpu
from jax.experimental.pallas import tpu_sc as plsc

DIM = 768
FF = 4 * DIM
E = 8
S = 2048
P = 2 * S
TOPK = 2
LANES = 128
TS2 = 128
G = P // TS2 + E
ROWS = G * TS2
NG = 32
GS = P // NG
NW = 32


def _gate_kernel(x_ref, wg_ref, bg_ref, sel_ref, aux_ref, w_ref, pos_ref,
                 texp_ref):
    x = x_ref[...]
    logits = jnp.dot(x, wg_ref[...], preferred_element_type=jnp.float32)
    logits = logits + bg_ref[...]
    col = lax.broadcasted_iota(jnp.int32, (S, LANES), 1)
    valid = col < E
    neg = jnp.float32(-jnp.inf)
    logits = jnp.where(valid, logits, neg)

    m = jnp.max(logits, axis=1, keepdims=True)
    ex = jnp.where(valid, jnp.exp(logits - m), 0.0)
    probs = ex / jnp.sum(ex, axis=1, keepdims=True)

    e0 = jnp.argmax(logits, axis=1)
    l0 = jnp.max(logits, axis=1)
    masked = jnp.where(col == e0[:, None], neg, logits)
    e1 = jnp.argmax(masked, axis=1)
    l1 = jnp.max(masked, axis=1)
    w0 = 1.0 / (1.0 + jnp.exp(l1 - l0))
    w1 = 1.0 - w0

    is0 = col == e0[:, None]
    is1 = col == e1[:, None]
    count_mask = ((is0 | is1) & valid).astype(jnp.float32)
    me = jnp.mean(probs, axis=0)
    ce = jnp.mean(count_mask, axis=0)
    aux_ref[...] = jnp.full((1, LANES), E * jnp.sum(me * ce), jnp.float32)
    sel_ref[...] = jnp.concatenate(
        [e0[:, None], e1[:, None]], axis=1).astype(jnp.int32)

    e_pair = jnp.concatenate([e0[:, None], e1[:, None]], axis=0)
    pcol = lax.broadcasted_iota(jnp.int32, (P, LANES), 1)
    onehot = (pcol == e_pair).astype(jnp.float32)

    ti = lax.broadcasted_iota(jnp.int32, (GS, GS), 0)
    tj = lax.broadcasted_iota(jnp.int32, (GS, GS), 1)
    tril = (tj <= ti).astype(jnp.float32)
    gsum = jnp.concatenate(
        [jnp.sum(onehot[g * GS:(g + 1) * GS], axis=0, keepdims=True)
         for g in range(NG)], axis=0)
    gi = lax.broadcasted_iota(jnp.int32, (NG, NG), 0)
    gj = lax.broadcasted_iota(jnp.int32, (NG, NG), 1)
    gtril = (gj < gi).astype(jnp.float32)
    gpre = jnp.dot(gtril, gsum, preferred_element_type=jnp.float32)
    counts = gpre[NG - 1:NG, :] + gsum[NG - 1:NG, :]

    ntiles = jnp.floor((counts + (TS2 - 1)) * (1.0 / TS2))
    ei = lax.broadcasted_iota(jnp.int32, (LANES, LANES), 0)
    ej = lax.broadcasted_iota(jnp.int32, (LANES, LANES), 1)
    upper = (ei < ej).astype(jnp.float32)
    tbase = jnp.dot(ntiles, upper, preferred_element_type=jnp.float32)
    base_rows = tbase * TS2

    rank = jnp.concatenate(
        [gpre[g:g + 1, :] +
         jnp.dot(tril, onehot[g * GS:(g + 1) * GS],
                 preferred_element_type=jnp.float32)
         for g in range(NG)], axis=0)
    pos = jnp.sum(onehot * (base_rows + rank - 1.0), axis=1, keepdims=True)

    ident = (ei == ej).astype(jnp.float32)
    pos_hi = jnp.floor(pos * (1.0 / 256.0))
    pos_lo = pos - pos_hi * 256.0

    def _transpose_cols(col):
        return jnp.concatenate(
            [lax.dot_general(col[g * GS:(g + 1) * GS], ident,
                             (((0,), (0,)), ((), ())),
                             preferred_element_type=jnp.float32)
             for g in range(NG)], axis=0)

    pos_rows = _transpose_cols(pos_hi) * 256.0 + _transpose_cols(pos_lo)
    pos_ref[...] = pos_rows.astype(jnp.int32)

    wpair = jnp.concatenate([w0[:, None], w1[:, None]], axis=0)
    w_ref[...] = jnp.broadcast_to(wpair, (P, LANES))

    tend = tbase + ntiles
    grow = ei.astype(jnp.float32)
    hit = jnp.where((ej < E) & (grow >= tend), 1.0, 0.0)
    texp = jnp.minimum(jnp.sum(hit, axis=1, keepdims=True), E - 1)
    texp_ref[...] = lax.dot_general(
        texp, ident, (((0,), (0,)), ((), ())),
        preferred_element_type=jnp.float32).astype(jnp.int32)


def _vmesh():
    return plsc.VectorSubcoreMesh(core_axis_name="c", subcore_axis_name="s")


def _sc_scatter_kernel(x_hbm, pos_hbm, w_hbm, xo_hbm, wo_hbm,
                       idx_v, rows_v, wrow_v):
    c = lax.axis_index("c")
    s = lax.axis_index("s")
    wid = s * 2 + c
    n = P // NW
    base = wid * n
    tok = lax.rem(base, S)
    pltpu.sync_copy(pos_hbm.at[wid], idx_v)
    pltpu.sync_copy(x_hbm.at[pl.ds(tok, n)], rows_v)
    pltpu.sync_copy(rows_v, xo_hbm.at[idx_v])
    pltpu.sync_copy(w_hbm.at[pl.ds(base, n)], wrow_v)
    pltpu.sync_copy(wrow_v, wo_hbm.at[idx_v])


def _sc_scatter(x2, pos_c, w_big):
    k = pl.kernel(
        _sc_scatter_kernel,
        out_type=[
            jax.ShapeDtypeStruct((ROWS, DIM), jnp.float32),
            jax.ShapeDtypeStruct((ROWS, LANES), jnp.float32),
        ],
        mesh=_vmesh(),
        scratch_types=[
            pltpu.VMEM((P // NW,), jnp.int32),
            pltpu.VMEM((P // NW, DIM), jnp.float32),
            pltpu.VMEM((P // NW, LANES), jnp.float32),
        ],
    )
    return k(x2, pos_c, w_big)


def _sc_gather_kernel(y_hbm, pos_hbm, o_hbm, idx_v, a_v, b_v):
    c = lax.axis_index("c")
    s = lax.axis_index("s")
    wid = s * 2 + c
    n = S // NW
    base = wid * n
    r = wid // 2
    cc = lax.rem(wid, 2) * n
    pltpu.sync_copy(pos_hbm.at[r, pl.ds(cc, n)], idx_v)
    pltpu.sync_copy(y_hbm.at[idx_v], a_v)
    pltpu.sync_copy(pos_hbm.at[NG // 2 + r, pl.ds(cc, n)], idx_v)
    pltpu.sync_copy(y_hbm.at[idx_v], b_v)

    @pl.loop(0, n)
    def _(i):
        @pl.loop(0, DIM, step=16)
        def _(j):
            a_v[i, pl.ds(j, 16)] = a_v[i, pl.ds(j, 16)] + b_v[i, pl.ds(j, 16)]

    pltpu.sync_copy(a_v, o_hbm.at[pl.ds(base, n)])


def _sc_gather_add(y_sorted, pos_c):
    k = pl.kernel(
        _sc_gather_kernel,
        out_type=jax.ShapeDtypeStruct((S, DIM), jnp.float32),
        mesh=_vmesh(),
        scratch_types=[
            pltpu.VMEM((S // NW,), jnp.int32),
            pltpu.VMEM((S // NW, DIM), jnp.float32),
            pltpu.VMEM((S // NW, DIM), jnp.float32),
        ],
    )
    return k(y_sorted, pos_c)


def _mlp_kernel(tmap_ref, x_ref, w1_ref, b1_ref, w2_ref, b2_ref, ws_ref,
                out_ref):
    h = jnp.dot(x_ref[...], w1_ref[0], preferred_element_type=jnp.float32)
    h = h + b1_ref[0]
    h = 0.5 * h * (1.0 + lax.erf(h * 0.7071067811865476))
    y = jnp.dot(h, w2_ref[0], preferred_element_type=jnp.float32)
    out_ref[...] = (y + b2_ref[0]) * ws_ref[:, 0:1]


def _grouped_mlp(tile_map, x_sorted, w_sorted, W1, b1, W2, b2):
    b1r = b1.reshape(E, 1, FF)
    b2r = b2.reshape(E, 1, DIM)
    grid_spec = pltpu.PrefetchScalarGridSpec(
        num_scalar_prefetch=1,
        grid=(G,),
        in_specs=[
            pl.BlockSpec((TS2, DIM), lambda g, m: (g, 0)),
            pl.BlockSpec((1, DIM, FF), lambda g, m: (m[0, g], 0, 0)),
            pl.BlockSpec((1, 1, FF), lambda g, m: (m[0, g], 0, 0)),
            pl.BlockSpec((1, FF, DIM), lambda g, m: (m[0, g], 0, 0)),
            pl.BlockSpec((1, 1, DIM), lambda g, m: (m[0, g], 0, 0)),
            pl.BlockSpec((TS2, LANES), lambda g, m: (g, 0)),
        ],
        out_specs=pl.BlockSpec((TS2, DIM), lambda g, m: (g, 0)),
    )
    return pl.pallas_call(
        _mlp_kernel,
        grid_spec=grid_spec,
        out_shape=jax.ShapeDtypeStruct((ROWS, DIM), jnp.float32),
        compiler_params=pltpu.CompilerParams(
            dimension_semantics=("arbitrary",),
        ),
    )(tile_map, x_sorted, W1, b1r, W2, b2r, w_sorted)


def kernel(x, W1, b1, W2, b2, Wg, bg):
    x2 = x.reshape(S, DIM)
    wg_p = jnp.pad(Wg, ((0, 0), (0, LANES - E)))
    bg_p = jnp.pad(bg, (0, LANES - E)).reshape(1, LANES)

    sel, aux, w_big, pos_c, tile_map = pl.pallas_call(
        _gate_kernel,
        out_shape=[
            jax.ShapeDtypeStruct((S, TOPK), jnp.int32),
            jax.ShapeDtypeStruct((1, LANES), jnp.float32),
            jax.ShapeDtypeStruct((P, LANES), jnp.float32),
            jax.ShapeDtypeStruct((NG, GS), jnp.int32),
            jax.ShapeDtypeStruct((1, LANES), jnp.int32),
        ],
    )(x2, wg_p, bg_p)

    x_sorted, w_sorted = _sc_scatter(x2, pos_c, w_big)
    y_sorted = _grouped_mlp(tile_map, x_sorted, w_sorted, W1, b1, W2, b2)
    out = _sc_gather_add(y_sorted, pos_c)

    output = out.reshape(1, S, DIM)
    selected = sel.reshape(1, S, TOPK)
    aux_loss = aux[0, 0]
    return (output, selected, aux_loss)


# speedup vs baseline: 1.0356x; 1.0356x over previous
import jax
import jax.numpy as jnp
from jax import lax
from jax.experimental import pallas as pl
from jax.experimental.pallas import tpu as pltpu
from jax.experimental.pallas import tpu_sc as plsc

DIM = 768
FF = 4 * DIM
E = 8
S = 2048
P = 2 * S
TOPK = 2
LANES = 128
TS2 = 256
G = P // TS2 + E
ROWS = G * TS2
NG = 32
GS = P // NG
NW = 32


def _gate_kernel(x_ref, wg_ref, bg_ref, sel_ref, aux_ref, w_ref, pos_ref,
                 texp_ref):
    x = x_ref[...]
    logits = jnp.dot(x, wg_ref[...], preferred_element_type=jnp.float32)
    logits = logits + bg_ref[...]
    col = lax.broadcasted_iota(jnp.int32, (S, LANES), 1)
    valid = col < E
    neg = jnp.float32(-jnp.inf)
    logits = jnp.where(valid, logits, neg)

    m = jnp.max(logits, axis=1, keepdims=True)
    ex = jnp.where(valid, jnp.exp(logits - m), 0.0)
    probs = ex / jnp.sum(ex, axis=1, keepdims=True)

    e0 = jnp.argmax(logits, axis=1)
    l0 = jnp.max(logits, axis=1)
    masked = jnp.where(col == e0[:, None], neg, logits)
    e1 = jnp.argmax(masked, axis=1)
    l1 = jnp.max(masked, axis=1)
    w0 = 1.0 / (1.0 + jnp.exp(l1 - l0))
    w1 = 1.0 - w0

    is0 = col == e0[:, None]
    is1 = col == e1[:, None]
    count_mask = ((is0 | is1) & valid).astype(jnp.float32)
    me = jnp.mean(probs, axis=0)
    ce = jnp.mean(count_mask, axis=0)
    aux_ref[...] = jnp.full((1, LANES), E * jnp.sum(me * ce), jnp.float32)
    sel_ref[...] = jnp.concatenate(
        [e0[:, None], e1[:, None]], axis=1).astype(jnp.int32)

    e_pair = jnp.concatenate([e0[:, None], e1[:, None]], axis=0)
    pcol = lax.broadcasted_iota(jnp.int32, (P, LANES), 1)
    onehot = (pcol == e_pair).astype(jnp.float32)

    ti = lax.broadcasted_iota(jnp.int32, (GS, GS), 0)
    tj = lax.broadcasted_iota(jnp.int32, (GS, GS), 1)
    tril = (tj <= ti).astype(jnp.float32)
    gsum = jnp.concatenate(
        [jnp.sum(onehot[g * GS:(g + 1) * GS], axis=0, keepdims=True)
         for g in range(NG)], axis=0)
    gi = lax.broadcasted_iota(jnp.int32, (NG, NG), 0)
    gj = lax.broadcasted_iota(jnp.int32, (NG, NG), 1)
    gtril = (gj < gi).astype(jnp.float32)
    gpre = jnp.dot(gtril, gsum, preferred_element_type=jnp.float32)
    counts = gpre[NG - 1:NG, :] + gsum[NG - 1:NG, :]

    ntiles = jnp.floor((counts + (TS2 - 1)) * (1.0 / TS2))
    ei = lax.broadcasted_iota(jnp.int32, (LANES, LANES), 0)
    ej = lax.broadcasted_iota(jnp.int32, (LANES, LANES), 1)
    upper = (ei < ej).astype(jnp.float32)
    tbase = jnp.dot(ntiles, upper, preferred_element_type=jnp.float32)
    base_rows = tbase * TS2

    rank = jnp.concatenate(
        [gpre[g:g + 1, :] +
         jnp.dot(tril, onehot[g * GS:(g + 1) * GS],
                 preferred_element_type=jnp.float32)
         for g in range(NG)], axis=0)
    pos = jnp.sum(onehot * (base_rows + rank - 1.0), axis=1, keepdims=True)

    ident = (ei == ej).astype(jnp.float32)
    pos_hi = jnp.floor(pos * (1.0 / 256.0))
    pos_lo = pos - pos_hi * 256.0

    def _transpose_cols(col):
        return jnp.concatenate(
            [lax.dot_general(col[g * GS:(g + 1) * GS], ident,
                             (((0,), (0,)), ((), ())),
                             preferred_element_type=jnp.float32)
             for g in range(NG)], axis=0)

    pos_rows = _transpose_cols(pos_hi) * 256.0 + _transpose_cols(pos_lo)
    pos_ref[...] = pos_rows.astype(jnp.int32)

    wpair = jnp.concatenate([w0[:, None], w1[:, None]], axis=0)
    w_ref[...] = jnp.broadcast_to(wpair, (P, LANES))

    tend = tbase + ntiles
    grow = ei.astype(jnp.float32)
    hit = jnp.where((ej < E) & (grow >= tend), 1.0, 0.0)
    texp = jnp.minimum(jnp.sum(hit, axis=1, keepdims=True), E - 1)
    texp_ref[...] = lax.dot_general(
        texp, ident, (((0,), (0,)), ((), ())),
        preferred_element_type=jnp.float32).astype(jnp.int32)


def _vmesh():
    return plsc.VectorSubcoreMesh(core_axis_name="c", subcore_axis_name="s")


def _sc_scatter_kernel(x_hbm, pos_hbm, w_hbm, xo_hbm, wo_hbm,
                       idx_v, rows_v, wrow_v):
    c = lax.axis_index("c")
    s = lax.axis_index("s")
    wid = s * 2 + c
    n = P // NW
    base = wid * n
    tok = lax.rem(base, S)
    pltpu.sync_copy(pos_hbm.at[wid], idx_v)
    pltpu.sync_copy(x_hbm.at[pl.ds(tok, n)], rows_v)
    pltpu.sync_copy(rows_v, xo_hbm.at[idx_v])
    pltpu.sync_copy(w_hbm.at[pl.ds(base, n)], wrow_v)
    pltpu.sync_copy(wrow_v, wo_hbm.at[idx_v])


def _sc_scatter(x2, pos_c, w_big):
    k = pl.kernel(
        _sc_scatter_kernel,
        out_type=[
            jax.ShapeDtypeStruct((ROWS, DIM), jnp.float32),
            jax.ShapeDtypeStruct((ROWS, LANES), jnp.float32),
        ],
        mesh=_vmesh(),
        scratch_types=[
            pltpu.VMEM((P // NW,), jnp.int32),
            pltpu.VMEM((P // NW, DIM), jnp.float32),
            pltpu.VMEM((P // NW, LANES), jnp.float32),
        ],
    )
    return k(x2, pos_c, w_big)


def _sc_gather_kernel(y_hbm, pos_hbm, o_hbm, idx_v, a_v, b_v):
    c = lax.axis_index("c")
    s = lax.axis_index("s")
    wid = s * 2 + c
    n = S // NW
    base = wid * n
    r = wid // 2
    cc = lax.rem(wid, 2) * n
    pltpu.sync_copy(pos_hbm.at[r, pl.ds(cc, n)], idx_v)
    pltpu.sync_copy(y_hbm.at[idx_v], a_v)
    pltpu.sync_copy(pos_hbm.at[NG // 2 + r, pl.ds(cc, n)], idx_v)
    pltpu.sync_copy(y_hbm.at[idx_v], b_v)

    @pl.loop(0, n)
    def _(i):
        @pl.loop(0, DIM, step=16)
        def _(j):
            a_v[i, pl.ds(j, 16)] = a_v[i, pl.ds(j, 16)] + b_v[i, pl.ds(j, 16)]

    pltpu.sync_copy(a_v, o_hbm.at[pl.ds(base, n)])


def _sc_gather_add(y_sorted, pos_c):
    k = pl.kernel(
        _sc_gather_kernel,
        out_type=jax.ShapeDtypeStruct((S, DIM), jnp.float32),
        mesh=_vmesh(),
        scratch_types=[
            pltpu.VMEM((S // NW,), jnp.int32),
            pltpu.VMEM((S // NW, DIM), jnp.float32),
            pltpu.VMEM((S // NW, DIM), jnp.float32),
        ],
    )
    return k(y_sorted, pos_c)


def _mlp_kernel(tmap_ref, x_ref, w1_ref, b1_ref, w2_ref, b2_ref, ws_ref,
                out_ref):
    h = jnp.dot(x_ref[...], w1_ref[0], preferred_element_type=jnp.float32)
    h = h + b1_ref[0]
    h = 0.5 * h * (1.0 + lax.erf(h * 0.7071067811865476))
    y = jnp.dot(h, w2_ref[0], preferred_element_type=jnp.float32)
    out_ref[...] = (y + b2_ref[0]) * ws_ref[:, 0:1]


def _grouped_mlp(tile_map, x_sorted, w_sorted, W1, b1, W2, b2):
    b1r = b1.reshape(E, 1, FF)
    b2r = b2.reshape(E, 1, DIM)
    grid_spec = pltpu.PrefetchScalarGridSpec(
        num_scalar_prefetch=1,
        grid=(G,),
        in_specs=[
            pl.BlockSpec((TS2, DIM), lambda g, m: (g, 0)),
            pl.BlockSpec((1, DIM, FF), lambda g, m: (m[0, g], 0, 0)),
            pl.BlockSpec((1, 1, FF), lambda g, m: (m[0, g], 0, 0)),
            pl.BlockSpec((1, FF, DIM), lambda g, m: (m[0, g], 0, 0)),
            pl.BlockSpec((1, 1, DIM), lambda g, m: (m[0, g], 0, 0)),
            pl.BlockSpec((TS2, LANES), lambda g, m: (g, 0)),
        ],
        out_specs=pl.BlockSpec((TS2, DIM), lambda g, m: (g, 0)),
    )
    return pl.pallas_call(
        _mlp_kernel,
        grid_spec=grid_spec,
        out_shape=jax.ShapeDtypeStruct((ROWS, DIM), jnp.float32),
        compiler_params=pltpu.CompilerParams(
            dimension_semantics=("arbitrary",),
        ),
    )(tile_map, x_sorted, W1, b1r, W2, b2r, w_sorted)


def kernel(x, W1, b1, W2, b2, Wg, bg):
    x2 = x.reshape(S, DIM)
    wg_p = jnp.pad(Wg, ((0, 0), (0, LANES - E)))
    bg_p = jnp.pad(bg, (0, LANES - E)).reshape(1, LANES)

    sel, aux, w_big, pos_c, tile_map = pl.pallas_call(
        _gate_kernel,
        out_shape=[
            jax.ShapeDtypeStruct((S, TOPK), jnp.int32),
            jax.ShapeDtypeStruct((1, LANES), jnp.float32),
            jax.ShapeDtypeStruct((P, LANES), jnp.float32),
            jax.ShapeDtypeStruct((NG, GS), jnp.int32),
            jax.ShapeDtypeStruct((1, LANES), jnp.int32),
        ],
    )(x2, wg_p, bg_p)

    x_sorted, w_sorted = _sc_scatter(x2, pos_c, w_big)
    y_sorted = _grouped_mlp(tile_map, x_sorted, w_sorted, W1, b1, W2, b2)
    out = _sc_gather_add(y_sorted, pos_c)

    output = out.reshape(1, S, DIM)
    selected = sel.reshape(1, S, TOPK)
    aux_loss = aux[0, 0]
    return (output, selected, aux_loss)


# speedup vs baseline: 1.0871x; 1.0497x over previous
import jax
import jax.numpy as jnp
from jax import lax
from jax.experimental import pallas as pl
from jax.experimental.pallas import tpu as pltpu
from jax.experimental.pallas import tpu_sc as plsc

DIM = 768
FF = 4 * DIM
E = 8
S = 2048
P = 2 * S
TOPK = 2
LANES = 128
TS2 = 256
G = P // TS2 + E
ROWS = G * TS2
NG = 32
GS = P // NG
NW = 32


def _gate_kernel(x_ref, wg_ref, bg_ref, sel_ref, aux_ref, w_ref, pos_ref,
                 texp_ref):
    x = x_ref[...]
    logits = jnp.dot(x, wg_ref[...], preferred_element_type=jnp.float32)
    logits = logits + bg_ref[...]
    col = lax.broadcasted_iota(jnp.int32, (S, LANES), 1)
    valid = col < E
    neg = jnp.float32(-jnp.inf)
    logits = jnp.where(valid, logits, neg)

    m = jnp.max(logits, axis=1, keepdims=True)
    ex = jnp.where(valid, jnp.exp(logits - m), 0.0)
    probs = ex / jnp.sum(ex, axis=1, keepdims=True)

    e0 = jnp.argmax(logits, axis=1)
    l0 = jnp.max(logits, axis=1)
    masked = jnp.where(col == e0[:, None], neg, logits)
    e1 = jnp.argmax(masked, axis=1)
    l1 = jnp.max(masked, axis=1)
    w0 = 1.0 / (1.0 + jnp.exp(l1 - l0))
    w1 = 1.0 - w0

    is0 = col == e0[:, None]
    is1 = col == e1[:, None]
    count_mask = ((is0 | is1) & valid).astype(jnp.float32)
    me = jnp.mean(probs, axis=0)
    ce = jnp.mean(count_mask, axis=0)
    aux_ref[...] = jnp.full((1, LANES), E * jnp.sum(me * ce), jnp.float32)
    sel_ref[...] = jnp.concatenate(
        [e0[:, None], e1[:, None]], axis=1).astype(jnp.int32)

    e_pair = jnp.concatenate([e0[:, None], e1[:, None]], axis=0)
    pcol = lax.broadcasted_iota(jnp.int32, (P, LANES), 1)
    onehot = (pcol == e_pair).astype(jnp.float32)

    ti = lax.broadcasted_iota(jnp.int32, (GS, GS), 0)
    tj = lax.broadcasted_iota(jnp.int32, (GS, GS), 1)
    tril = (tj <= ti).astype(jnp.float32)
    gsum = jnp.concatenate(
        [jnp.sum(onehot[g * GS:(g + 1) * GS], axis=0, keepdims=True)
         for g in range(NG)], axis=0)
    gi = lax.broadcasted_iota(jnp.int32, (NG, NG), 0)
    gj = lax.broadcasted_iota(jnp.int32, (NG, NG), 1)
    gtril = (gj < gi).astype(jnp.float32)
    gpre = jnp.dot(gtril, gsum, preferred_element_type=jnp.float32)
    counts = gpre[NG - 1:NG, :] + gsum[NG - 1:NG, :]

    ntiles = jnp.floor((counts + (TS2 - 1)) * (1.0 / TS2))
    ei = lax.broadcasted_iota(jnp.int32, (LANES, LANES), 0)
    ej = lax.broadcasted_iota(jnp.int32, (LANES, LANES), 1)
    upper = (ei < ej).astype(jnp.float32)
    tbase = jnp.dot(ntiles, upper, preferred_element_type=jnp.float32)
    base_rows = tbase * TS2

    rank = jnp.concatenate(
        [gpre[g:g + 1, :] +
         jnp.dot(tril, onehot[g * GS:(g + 1) * GS],
                 preferred_element_type=jnp.float32)
         for g in range(NG)], axis=0)
    pos = jnp.sum(onehot * (base_rows + rank - 1.0), axis=1, keepdims=True)

    ident = (ei == ej).astype(jnp.float32)
    pos_hi = jnp.floor(pos * (1.0 / 256.0))
    pos_lo = pos - pos_hi * 256.0

    def _transpose_cols(col):
        return jnp.concatenate(
            [lax.dot_general(col[g * GS:(g + 1) * GS], ident,
                             (((0,), (0,)), ((), ())),
                             preferred_element_type=jnp.float32)
             for g in range(NG)], axis=0)

    pos_rows = _transpose_cols(pos_hi) * 256.0 + _transpose_cols(pos_lo)
    pos_ref[...] = pos_rows.astype(jnp.int32)

    wpair = jnp.concatenate([w0[:, None], w1[:, None]], axis=0)
    w_ref[...] = jnp.broadcast_to(wpair, (P, LANES))

    tend = tbase + ntiles
    grow = ei.astype(jnp.float32)
    hit = jnp.where((ej < E) & (grow >= tend), 1.0, 0.0)
    texp = jnp.minimum(jnp.sum(hit, axis=1, keepdims=True), E - 1)
    total_tiles = jnp.sum(jnp.where(ej[0:1, :] < E, ntiles, 0.0))
    live = jnp.where(grow[:, 0:1] < total_tiles, 1.0, 0.0)
    texp_ref[...] = jnp.concatenate(
        [lax.dot_general(c, ident, (((0,), (0,)), ((), ())),
                         preferred_element_type=jnp.float32)
         for c in (texp, live)], axis=0).astype(jnp.int32)


def _vmesh():
    return plsc.VectorSubcoreMesh(core_axis_name="c", subcore_axis_name="s")


def _sc_scatter_kernel(x_hbm, pos_hbm, w_hbm, xo_hbm, wo_hbm,
                       idx_v, rows_v, wrow_v):
    c = lax.axis_index("c")
    s = lax.axis_index("s")
    wid = s * 2 + c
    n = P // NW
    base = wid * n
    tok = lax.rem(base, S)
    pltpu.sync_copy(pos_hbm.at[wid], idx_v)
    pltpu.sync_copy(x_hbm.at[pl.ds(tok, n)], rows_v)
    pltpu.sync_copy(rows_v, xo_hbm.at[idx_v])
    pltpu.sync_copy(w_hbm.at[pl.ds(base, n)], wrow_v)
    pltpu.sync_copy(wrow_v, wo_hbm.at[idx_v])


def _sc_scatter(x2, pos_c, w_big):
    k = pl.kernel(
        _sc_scatter_kernel,
        out_type=[
            jax.ShapeDtypeStruct((ROWS, DIM), jnp.float32),
            jax.ShapeDtypeStruct((ROWS, LANES), jnp.float32),
        ],
        mesh=_vmesh(),
        scratch_types=[
            pltpu.VMEM((P // NW,), jnp.int32),
            pltpu.VMEM((P // NW, DIM), jnp.float32),
            pltpu.VMEM((P // NW, LANES), jnp.float32),
        ],
    )
    return k(x2, pos_c, w_big)


def _sc_gather_kernel(y_hbm, pos_hbm, o_hbm, idx_v, a_v, b_v):
    c = lax.axis_index("c")
    s = lax.axis_index("s")
    wid = s * 2 + c
    n = S // NW
    base = wid * n
    r = wid // 2
    cc = lax.rem(wid, 2) * n
    pltpu.sync_copy(pos_hbm.at[r, pl.ds(cc, n)], idx_v)
    pltpu.sync_copy(y_hbm.at[idx_v], a_v)
    pltpu.sync_copy(pos_hbm.at[NG // 2 + r, pl.ds(cc, n)], idx_v)
    pltpu.sync_copy(y_hbm.at[idx_v], b_v)

    @pl.loop(0, n)
    def _(i):
        @pl.loop(0, DIM, step=16)
        def _(j):
            a_v[i, pl.ds(j, 16)] = a_v[i, pl.ds(j, 16)] + b_v[i, pl.ds(j, 16)]

    pltpu.sync_copy(a_v, o_hbm.at[pl.ds(base, n)])


def _sc_gather_add(y_sorted, pos_c):
    k = pl.kernel(
        _sc_gather_kernel,
        out_type=jax.ShapeDtypeStruct((S, DIM), jnp.float32),
        mesh=_vmesh(),
        scratch_types=[
            pltpu.VMEM((S // NW,), jnp.int32),
            pltpu.VMEM((S // NW, DIM), jnp.float32),
            pltpu.VMEM((S // NW, DIM), jnp.float32),
        ],
    )
    return k(y_sorted, pos_c)


def _mlp_kernel(tmap_ref, x_ref, w1_ref, b1_ref, w2_ref, b2_ref, ws_ref,
                out_ref):
    g = pl.program_id(0)

    @pl.when(tmap_ref[1, g] > 0)
    def _():
        h = jnp.dot(x_ref[...], w1_ref[0], preferred_element_type=jnp.float32)
        h = h + b1_ref[0]
        h = 0.5 * h * (1.0 + lax.erf(h * 0.7071067811865476))
        y = jnp.dot(h, w2_ref[0], preferred_element_type=jnp.float32)
        out_ref[...] = (y + b2_ref[0]) * ws_ref[:, 0:1]


def _grouped_mlp(tile_map, x_sorted, w_sorted, W1, b1, W2, b2):
    b1r = b1.reshape(E, 1, FF)
    b2r = b2.reshape(E, 1, DIM)
    grid_spec = pltpu.PrefetchScalarGridSpec(
        num_scalar_prefetch=1,
        grid=(G,),
        in_specs=[
            pl.BlockSpec((TS2, DIM), lambda g, m: (g, 0)),
            pl.BlockSpec((1, DIM, FF), lambda g, m: (m[0, g], 0, 0)),
            pl.BlockSpec((1, 1, FF), lambda g, m: (m[0, g], 0, 0)),
            pl.BlockSpec((1, FF, DIM), lambda g, m: (m[0, g], 0, 0)),
            pl.BlockSpec((1, 1, DIM), lambda g, m: (m[0, g], 0, 0)),
            pl.BlockSpec((TS2, LANES), lambda g, m: (g, 0)),
        ],
        out_specs=pl.BlockSpec((TS2, DIM), lambda g, m: (g, 0)),
    )
    return pl.pallas_call(
        _mlp_kernel,
        grid_spec=grid_spec,
        out_shape=jax.ShapeDtypeStruct((ROWS, DIM), jnp.float32),
        compiler_params=pltpu.CompilerParams(
            dimension_semantics=("arbitrary",),
        ),
    )(tile_map, x_sorted, W1, b1r, W2, b2r, w_sorted)


def kernel(x, W1, b1, W2, b2, Wg, bg):
    x2 = x.reshape(S, DIM)
    wg_p = jnp.pad(Wg, ((0, 0), (0, LANES - E)))
    bg_p = jnp.pad(bg, (0, LANES - E)).reshape(1, LANES)

    sel, aux, w_big, pos_c, tile_map = pl.pallas_call(
        _gate_kernel,
        out_shape=[
            jax.ShapeDtypeStruct((S, TOPK), jnp.int32),
            jax.ShapeDtypeStruct((1, LANES), jnp.float32),
            jax.ShapeDtypeStruct((P, LANES), jnp.float32),
            jax.ShapeDtypeStruct((NG, GS), jnp.int32),
            jax.ShapeDtypeStruct((2, LANES), jnp.int32),
        ],
    )(x2, wg_p, bg_p)

    x_sorted, w_sorted = _sc_scatter(x2, pos_c, w_big)
    y_sorted = _grouped_mlp(tile_map, x_sorted, w_sorted, W1, b1, W2, b2)
    out = _sc_gather_add(y_sorted, pos_c)

    output = out.reshape(1, S, DIM)
    selected = sel.reshape(1, S, TOPK)
    aux_loss = aux[0, 0]
    return (output, selected, aux_loss)
